# baseline (speedup 1.0000x reference)
import functools

import jax
import jax.numpy as jnp
from jax import lax
from jax.experimental import pallas as pl
from jax.experimental.pallas import tpu as pltpu
from jax.experimental.pallas import tpu_sc as plsc

_PI = 3.141592653589793
_HIDDEN = 64
_GAMMA = 12.0
_EPSILON = 2.0
_EMB_RANGE = (_GAMMA + _EPSILON) / _HIDDEN
_PHASE_SCALE = _PI / _EMB_RANGE

_B = 16384
_POS = 1024
_NC = 2
_NS = 16
_NW = _NC * _NS
_BPW = _B // _NW
_CHUNK = 128
_NCHUNK = _BPW // _CHUNK


def _gather_body(h_idx, t_idx, r_idx, ent, rel2, out_h, out_t, out_r,
                 idx_v, rows_v, sem):
    wid = lax.axis_index("s") * _NC + lax.axis_index("c")
    base = wid * _BPW
    row0 = wid * _NCHUNK

    def gather_table(idx_hbm, table, out_hbm):
        pltpu.sync_copy(idx_hbm.at[pl.ds(row0, _NCHUNK)], idx_v)
        copies = [
            pltpu.async_copy(table.at[idx_v.at[j]],
                             rows_v.at[pl.ds(j * _CHUNK, _CHUNK)], sem)
            for j in range(_NCHUNK)
        ]
        for c in copies:
            c.wait()
        pltpu.sync_copy(rows_v, out_hbm.at[pl.ds(base, _BPW)])

    gather_table(h_idx, ent, out_h)
    gather_table(t_idx, ent, out_t)
    gather_table(r_idx, rel2, out_r)


_gather = functools.partial(
    pl.kernel,
    mesh=plsc.VectorSubcoreMesh(core_axis_name="c", subcore_axis_name="s"),
    out_type=(
        jax.ShapeDtypeStruct((_B, 2 * _HIDDEN), jnp.float32),
        jax.ShapeDtypeStruct((_B, 2 * _HIDDEN), jnp.float32),
        jax.ShapeDtypeStruct((_B, 2 * _HIDDEN), jnp.float32),
    ),
    scratch_types=[
        pltpu.VMEM((_NCHUNK, _CHUNK), jnp.int32),
        pltpu.VMEM((_BPW, 2 * _HIDDEN), jnp.float32),
        pltpu.SemaphoreType.DMA,
    ],
)(_gather_body)


def _score_body(h_ref, t_ref, rp_ref, par_ref, o_ref):
    h = h_ref[...]
    t = t_ref[...]
    rp = rp_ref[...]
    par = par_ref[...]
    re_h = h[:, :_HIDDEN]
    im_h = h[:, _HIDDEN:]
    re_t = t[:, :_HIDDEN]
    im_t = t[:, _HIDDEN:]
    r = jnp.where(par == 0, rp[:, :_HIDDEN], rp[:, _HIDDEN:])
    ph = r * _PHASE_SCALE
    c = jnp.cos(ph)
    s = jnp.sin(ph)
    re = re_h * c - im_h * s - re_t
    im = re_h * s + im_h * c - im_t
    v = jnp.sqrt(re * re + im * im)
    o_ref[...] = _GAMMA - jnp.sum(v, axis=1, keepdims=True)


_SCORE_BLK = 2048


def _score(h_rows, t_rows, rp_rows, parity):
    return pl.pallas_call(
        _score_body,
        grid=(_B // _SCORE_BLK,),
        in_specs=[
            pl.BlockSpec((_SCORE_BLK, 2 * _HIDDEN), lambda i: (i, 0)),
            pl.BlockSpec((_SCORE_BLK, 2 * _HIDDEN), lambda i: (i, 0)),
            pl.BlockSpec((_SCORE_BLK, 2 * _HIDDEN), lambda i: (i, 0)),
            pl.BlockSpec((_SCORE_BLK, 1), lambda i: (i, 0)),
        ],
        out_specs=pl.BlockSpec((_SCORE_BLK, 1), lambda i: (i, 0)),
        out_shape=jax.ShapeDtypeStruct((_B, 1), jnp.float32),
    )(h_rows, t_rows, rp_rows, parity)


def kernel(input, ent_emb, rel_emb):
    h_idx = input[:, 0].reshape(_B // _CHUNK, _CHUNK)
    t_idx = input[:, 2].reshape(_B // _CHUNK, _CHUNK)
    r_col = input[:, 1]
    r_pair = (r_col >> 1).reshape(_B // _CHUNK, _CHUNK)
    parity = (r_col & 1).reshape(_B, 1)
    rel2 = rel_emb.reshape(rel_emb.shape[0] // 2, 2 * _HIDDEN)
    h_rows, t_rows, rp_rows = _gather(h_idx, t_idx, r_pair, ent_emb, rel2)
    scores = _score(h_rows, t_rows, rp_rows, parity)
    return scores[:_POS], scores[_POS:]


# speedup vs baseline: 1.1065x; 1.1065x over previous
import functools

import jax
import jax.numpy as jnp
from jax import lax
from jax.experimental import pallas as pl
from jax.experimental.pallas import tpu as pltpu
from jax.experimental.pallas import tpu_sc as plsc

_PI = 3.141592653589793
_HIDDEN = 64
_GAMMA = 12.0
_EPSILON = 2.0
_EMB_RANGE = (_GAMMA + _EPSILON) / _HIDDEN
_PHASE_SCALE = _PI / _EMB_RANGE

_B = 16384
_POS = 1024
_NC = 2
_NS = 16
_NW = _NC * _NS
_BPW = _B // _NW
_CHUNK = 128
_NCHUNK = _BPW // _CHUNK

_TWO_OVER_PI = 0.6366197723675814
_PIO2_HI = 1.5707963705062866
_PIO2_LO = -4.371139000186241e-08
_S1, _S2, _S3 = -1.0 / 6.0, 1.0 / 120.0, -1.0 / 5040.0
_C1, _C2, _C3, _C4 = -0.5, 1.0 / 24.0, -1.0 / 720.0, 1.0 / 40320.0


def _gather_body(cols, ent, rel2, out_h, out_t, out_r,
                 hidx_v, tidx_v, ridx_v, rows_v, sem):
    wid = lax.axis_index("s") * _NC + lax.axis_index("c")
    base = wid * _BPW

    for j in range(_NCHUNK):
        pltpu.sync_copy(cols.at[0, 0, pl.ds(base + j * _CHUNK, _CHUNK)],
                        hidx_v.at[j])
        pltpu.sync_copy(cols.at[1, 0, pl.ds(base + j * _CHUNK, _CHUNK)],
                        tidx_v.at[j])
        pltpu.sync_copy(cols.at[2, 0, pl.ds(base + j * _CHUNK, _CHUNK)],
                        ridx_v.at[j])

    def gather_table(idx_v, table, dst):
        copies = [
            pltpu.async_copy(table.at[idx_v.at[j]],
                             dst.at[pl.ds(j * _CHUNK, _CHUNK)], sem)
            for j in range(_NCHUNK)
        ]
        for c in copies:
            c.wait()

    gather_table(hidx_v, ent, rows_v)
    pltpu.sync_copy(rows_v, out_h.at[pl.ds(base, _BPW)])
    gather_table(tidx_v, ent, rows_v)
    pltpu.sync_copy(rows_v, out_t.at[pl.ds(base, _BPW)])
    gather_table(ridx_v, rel2, rows_v)
    pltpu.sync_copy(rows_v, out_r.at[pl.ds(base, _BPW)])


@functools.lru_cache(maxsize=1)
def _make_gather():
  return functools.partial(
    pl.kernel,
    mesh=plsc.VectorSubcoreMesh(core_axis_name="c", subcore_axis_name="s"),
    out_type=(
        jax.ShapeDtypeStruct((_B, 2 * _HIDDEN), jnp.float32),
        jax.ShapeDtypeStruct((_B, 2 * _HIDDEN), jnp.float32),
        jax.ShapeDtypeStruct((_B, 2 * _HIDDEN), jnp.float32),
    ),
    scratch_types=[
        pltpu.VMEM((_NCHUNK, _CHUNK), jnp.int32),
        pltpu.VMEM((_NCHUNK, _CHUNK), jnp.int32),
        pltpu.VMEM((_NCHUNK, _CHUNK), jnp.int32),
        pltpu.VMEM((_BPW, 2 * _HIDDEN), jnp.float32),
        pltpu.SemaphoreType.DMA,
    ],
  )(_gather_body)


def _sincos(ph):
    half = jnp.where(ph < 0.0, -0.5, 0.5)
    k = (ph * _TWO_OVER_PI + half).astype(jnp.int32)
    kf = k.astype(jnp.float32)
    r = ph - kf * _PIO2_HI - kf * _PIO2_LO
    z = r * r
    s_r = r * (1.0 + z * (_S1 + z * (_S2 + z * _S3)))
    c_r = 1.0 + z * (_C1 + z * (_C2 + z * (_C3 + z * _C4)))
    swap = (k & 1) == 1
    sign_s = jnp.where((k & 2) == 2, -1.0, 1.0)
    sign_c = jnp.where(((k + 1) & 2) == 2, -1.0, 1.0)
    s = sign_s * jnp.where(swap, c_r, s_r)
    c = sign_c * jnp.where(swap, s_r, c_r)
    return s, c


_SCORE_BLK = 1024


def _score_body(h_ref, t_ref, rp_ref, par_ref, p_ref, n_ref):
    i = pl.program_id(0)
    h = h_ref[...]
    t = t_ref[...]
    rp = rp_ref[...]
    par = par_ref[...]
    re_h = h[:, :_HIDDEN]
    im_h = h[:, _HIDDEN:]
    re_t = t[:, :_HIDDEN]
    im_t = t[:, _HIDDEN:]
    r = jnp.where(par == 0, rp[:, :_HIDDEN], rp[:, _HIDDEN:])
    s, c = _sincos(r * _PHASE_SCALE)
    re = re_h * c - im_h * s - re_t
    im = re_h * s + im_h * c - im_t
    v = jnp.sqrt(re * re + im * im)
    res = _GAMMA - jnp.sum(v, axis=1, keepdims=True)

    @pl.when(i == 0)
    def _():
        p_ref[...] = res

    @pl.when(i > 0)
    def _():
        n_ref[...] = res


def _score(h_rows, t_rows, rp_rows, parity):
    nblk = _B // _SCORE_BLK
    return pl.pallas_call(
        _score_body,
        grid=(nblk,),
        in_specs=[
            pl.BlockSpec((_SCORE_BLK, 2 * _HIDDEN), lambda i: (i, 0)),
            pl.BlockSpec((_SCORE_BLK, 2 * _HIDDEN), lambda i: (i, 0)),
            pl.BlockSpec((_SCORE_BLK, 2 * _HIDDEN), lambda i: (i, 0)),
            pl.BlockSpec((_SCORE_BLK, 1), lambda i: (i, 0)),
        ],
        out_specs=[
            pl.BlockSpec((_POS, 1), lambda i: (0, 0)),
            pl.BlockSpec((_SCORE_BLK, 1),
                         lambda i: (jnp.maximum(i - 1, 0), 0)),
        ],
        out_shape=[
            jax.ShapeDtypeStruct((_POS, 1), jnp.float32),
            jax.ShapeDtypeStruct((_B - _POS, 1), jnp.float32),
        ],
    )(h_rows, t_rows, rp_rows, parity)


def kernel(input, ent_emb, rel_emb):
    h_col = input[:, 0]
    r_col = input[:, 1]
    t_col = input[:, 2]
    cols = jnp.stack([h_col, t_col, r_col >> 1]).reshape(3, 1, _B)
    parity = (r_col & 1).reshape(_B, 1)
    rel2 = rel_emb.reshape(rel_emb.shape[0] // 2, 2 * _HIDDEN)
    h_rows, t_rows, rp_rows = _make_gather()(cols, ent_emb, rel2)
    p_score, n_score = _score(h_rows, t_rows, rp_rows, parity)
    return p_score, n_score
